# baseline (speedup 1.0000x reference)
import functools

import jax
import jax.numpy as jnp
from jax import lax
from jax.experimental import pallas as pl
from jax.experimental.pallas import tpu as pltpu
from jax.experimental.pallas import tpu_sc as plsc

_D = 1000
_N = 1024 * 50
_NC = 2
_NS = 16
_NW = _NC * _NS
_RPW = _N // _NW
_CH = 40
_NCHUNK = _RPW // _CH


def _body(x_hbm, table_hbm, out_hbm, idx_v, buf0, buf1, sem0, sem1):
    wid = lax.axis_index("s") * _NC + lax.axis_index("c")
    base = pl.multiple_of(wid * _RPW, _RPW)
    pltpu.sync_copy(x_hbm.at[pl.ds(base, _RPW)], idx_v)

    def start(cc, buf, sem):
        off = pl.multiple_of(cc * _CH, _CH)
        pltpu.async_copy(table_hbm.at[idx_v.at[pl.ds(off, _CH)]], buf, sem)

    def wait(buf, sem):
        pltpu.make_async_copy(table_hbm.at[pl.ds(0, _CH)], buf, sem).wait()

    start(0, buf0, sem0)

    @pl.loop(0, _NCHUNK, step=2)
    def _(c):
        for p, (buf, sem), (nbuf, nsem) in (
            (0, (buf0, sem0), (buf1, sem1)),
            (1, (buf1, sem1), (buf0, sem0)),
        ):
            cc = c + p

            @pl.when(cc + 1 < _NCHUNK)
            def _():
                start(cc + 1, nbuf, nsem)

            wait(buf, sem)
            row0 = pl.multiple_of(base + cc * _CH, _CH)
            pltpu.sync_copy(buf, out_hbm.at[pl.ds(row0, _CH)])


@functools.partial(
    pl.kernel,
    out_type=jax.ShapeDtypeStruct((_N, _D), jnp.float32),
    mesh=plsc.VectorSubcoreMesh(core_axis_name="c", subcore_axis_name="s"),
    scratch_types=[
        pltpu.VMEM((_RPW,), jnp.int32),
        pltpu.VMEM((_CH, _D), jnp.float32),
        pltpu.VMEM((_CH, _D), jnp.float32),
        pltpu.SemaphoreType.DMA,
        pltpu.SemaphoreType.DMA,
    ],
    compiler_params=pltpu.CompilerParams(use_tc_tiling_on_sc=False),
)
def _gather(x_hbm, table_hbm, out_hbm, idx_v, buf0, buf1, sem0, sem1):
    _body(x_hbm, table_hbm, out_hbm, idx_v, buf0, buf1, sem0, sem1)


def kernel(X, table):
    xf = X.reshape(-1).astype(jnp.int32)
    return _gather(xf, table)


# speedup vs baseline: 1.1345x; 1.1345x over previous
import functools

import jax
import jax.numpy as jnp
from jax import lax
from jax.experimental import pallas as pl
from jax.experimental.pallas import tpu as pltpu
from jax.experimental.pallas import tpu_sc as plsc

_D = 1000
_N = 1024 * 50
_NC = 2
_NS = 16
_NW = _NC * _NS
_RPW = _N // _NW
_CH = 32
_NCHUNK = _RPW // _CH


def _body(x_hbm, table_hbm, out_hbm, idx_v, tab_sh, buf0, buf1, sem0, sem1):
    sid = lax.axis_index("s")
    wid = sid * _NC + lax.axis_index("c")
    base = pl.multiple_of(wid * _RPW, _RPW)
    pltpu.sync_copy(x_hbm.at[pl.ds(base, _RPW)], idx_v)

    @pl.when(sid < 8)
    def _():
        r0 = pl.multiple_of(sid * 125, 125)
        pltpu.sync_copy(table_hbm.at[pl.ds(r0, 125)], tab_sh.at[pl.ds(r0, 125)])

    plsc.subcore_barrier()

    def start(cc, buf, sem):
        off = pl.multiple_of(cc * _CH, _CH)
        pltpu.async_copy(tab_sh.at[idx_v.at[pl.ds(off, _CH)]], buf, sem)

    def wait(buf, sem):
        pltpu.make_async_copy(table_hbm.at[pl.ds(0, _CH)], buf, sem).wait()

    start(0, buf0, sem0)

    @pl.loop(0, _NCHUNK, step=2)
    def _(c):
        for p, (buf, sem), (nbuf, nsem) in (
            (0, (buf0, sem0), (buf1, sem1)),
            (1, (buf1, sem1), (buf0, sem0)),
        ):
            cc = c + p

            @pl.when(cc + 1 < _NCHUNK)
            def _():
                start(cc + 1, nbuf, nsem)

            wait(buf, sem)
            row0 = pl.multiple_of(base + cc * _CH, _CH)
            pltpu.sync_copy(buf, out_hbm.at[pl.ds(row0, _CH)])


@functools.partial(
    pl.kernel,
    out_type=jax.ShapeDtypeStruct((_N, _D), jnp.float32),
    mesh=plsc.VectorSubcoreMesh(core_axis_name="c", subcore_axis_name="s"),
    scratch_types=[
        pltpu.VMEM((_RPW,), jnp.int32),
        pltpu.VMEM_SHARED((1000, _D), jnp.float32),
        pltpu.VMEM((_CH, _D), jnp.float32),
        pltpu.VMEM((_CH, _D), jnp.float32),
        pltpu.SemaphoreType.DMA,
        pltpu.SemaphoreType.DMA,
    ],
    compiler_params=pltpu.CompilerParams(use_tc_tiling_on_sc=False),
)
def _gather(x_hbm, table_hbm, out_hbm, idx_v, tab_sh, buf0, buf1, sem0, sem1):
    _body(x_hbm, table_hbm, out_hbm, idx_v, tab_sh, buf0, buf1, sem0, sem1)


def kernel(X, table):
    xf = X.reshape(-1).astype(jnp.int32)
    return _gather(xf, table)


# speedup vs baseline: 1.4696x; 1.2953x over previous
import functools

import jax
import jax.numpy as jnp
from jax import lax
from jax.experimental import pallas as pl
from jax.experimental.pallas import tpu as pltpu
from jax.experimental.pallas import tpu_sc as plsc

_V = 1000
_D = 1000
_DP = 1024
_N = 1024 * 50
_NC = 2
_NS = 16
_NW = _NC * _NS
_RPW = _N // _NW
_CH = 32
_SUB = 16
_NCHUNK = _RPW // _CH


def _trim_row(buf, sr, ob, r):
    for c in range(0, _D - 16, 16):
        ob[r, pl.ds(c, 16)] = buf[sr, pl.ds(c, 16)]
    ob[r, pl.ds(_D - 16, 16)] = buf[sr, pl.ds(_D - 16, 16)]


def _body(x_hbm, table_hbm, out_hbm, idx_v, buf0, buf1, ob0, ob1,
          sem0, sem1, osem0, osem1):
    sid = lax.axis_index("s")
    wid = sid * _NC + lax.axis_index("c")
    base = pl.multiple_of(wid * _RPW, 8)
    pltpu.sync_copy(x_hbm.at[wid, 0], idx_v)

    def chunk_idx(cc):
        return idx_v.at[pl.ds(pl.multiple_of(cc * _CH, _CH), _CH)]

    def start(cc, buf, sem):
        pltpu.async_copy(table_hbm.at[chunk_idx(cc)], buf, sem)

    def wait(cc, buf, sem):
        pltpu.make_async_copy(table_hbm.at[chunk_idx(cc)], buf, sem).wait()

    start(0, buf0, sem0)

    @pl.loop(0, _NCHUNK, step=2)
    def _(c):
        for p, (buf, sem), (nbuf, nsem) in (
            (0, (buf0, sem0), (buf1, sem1)),
            (1, (buf1, sem1), (buf0, sem0)),
        ):
            cc = c + p

            @pl.when(cc + 1 < _NCHUNK)
            def _():
                start(cc + 1, nbuf, nsem)

            wait(cc, buf, sem)

            for sub, ob, osem in ((0, ob0, osem0), (1, ob1, osem1)):
                row0 = pl.multiple_of(base + cc * _CH + sub * _SUB, 8)
                orows = out_hbm.at[pl.ds(row0, _SUB)]

                @pl.when(cc > 0)
                def _():
                    pltpu.make_async_copy(ob, orows, osem).wait()

                @pl.loop(0, _SUB)
                def _(r):
                    _trim_row(buf, sub * _SUB + r, ob, r)

                pltpu.async_copy(ob, orows, osem)

    lastrow = pl.multiple_of(base + _RPW - _SUB, 8)
    pltpu.make_async_copy(ob0, out_hbm.at[pl.ds(lastrow, _SUB)], osem0).wait()
    pltpu.make_async_copy(ob1, out_hbm.at[pl.ds(lastrow, _SUB)], osem1).wait()


@functools.partial(
    pl.kernel,
    out_type=jax.ShapeDtypeStruct((_N, _D), jnp.float32),
    mesh=plsc.VectorSubcoreMesh(core_axis_name="c", subcore_axis_name="s"),
    scratch_types=[
        pltpu.VMEM((_RPW,), jnp.int32),
        pltpu.VMEM((_CH, _DP), jnp.float32),
        pltpu.VMEM((_CH, _DP), jnp.float32),
        pltpu.VMEM((_SUB, _D), jnp.float32),
        pltpu.VMEM((_SUB, _D), jnp.float32),
        pltpu.SemaphoreType.DMA,
        pltpu.SemaphoreType.DMA,
        pltpu.SemaphoreType.DMA,
        pltpu.SemaphoreType.DMA,
    ],
)
def _gather(x_hbm, table_hbm, out_hbm, idx_v, buf0, buf1, ob0, ob1,
            sem0, sem1, osem0, osem1):
    _body(x_hbm, table_hbm, out_hbm, idx_v, buf0, buf1, ob0, ob1,
          sem0, sem1, osem0, osem1)


def kernel(X, table):
    xf = X.reshape(_NW, 1, _RPW).astype(jnp.int32)
    tp = jnp.pad(table, ((0, 0), (0, _DP - _D)))
    return _gather(xf, tp)
